# baseline (speedup 1.0000x reference)
import functools

import jax
import jax.numpy as jnp
from jax import lax
from jax.experimental import pallas as pl
from jax.experimental.pallas import tpu as pltpu
from jax.experimental.pallas import tpu_sc as plsc

N_TASKS = 16
N_MODALITIES = 8
PRIOR_LEN = 16
EMBED_DIM = 768
BATCH = 1024

ROW = PRIOR_LEN * EMBED_DIM
NROWS = 2 * BATCH
NW = 32
ROWS_PER_W = NROWS // NW
CB = 4
NCHUNK = ROWS_PER_W // CB


def _gather_body(table_hbm, idx_hbm, out_hbm, idx_v, buf, sem):
    nc = 2
    wid = lax.axis_index("s") * nc + lax.axis_index("c")
    pltpu.sync_copy(idx_hbm.at[wid], idx_v)

    def chunk(g, carry):
        pltpu.async_copy(table_hbm.at[idx_v.at[g]], buf, sem).wait()
        pltpu.sync_copy(buf, out_hbm.at[pl.ds(wid * ROWS_PER_W + g * CB, CB)])
        return carry

    lax.fori_loop(0, NCHUNK, chunk, 0)


@jax.jit
def _gather(table, src_idx):
    mesh = plsc.VectorSubcoreMesh(core_axis_name="c", subcore_axis_name="s")
    f = pl.kernel(
        _gather_body,
        out_type=jax.ShapeDtypeStruct((NROWS, ROW), jnp.float32),
        mesh=mesh,
        scratch_types=[
            pltpu.VMEM((NCHUNK, CB), jnp.int32),
            pltpu.VMEM((CB, ROW), jnp.float32),
            pltpu.SemaphoreType.DMA,
        ],
    )
    return f(table, src_idx)


def kernel(task_table, modality_table, task_idx, modality_idx):
    table = jnp.concatenate(
        [task_table.reshape(N_TASKS, ROW),
         modality_table.reshape(N_MODALITIES, ROW)], axis=0)
    src_idx = jnp.stack(
        [task_idx.astype(jnp.int32),
         modality_idx.astype(jnp.int32) + N_TASKS], axis=1)
    src_idx = src_idx.reshape(NW, NCHUNK, CB)
    out = _gather(table, src_idx)
    return out.reshape(BATCH, 2 * PRIOR_LEN, EMBED_DIM)


# speedup vs baseline: 1.0126x; 1.0126x over previous
import functools

import jax
import jax.numpy as jnp
from jax import lax
from jax.experimental import pallas as pl
from jax.experimental.pallas import tpu as pltpu
from jax.experimental.pallas import tpu_sc as plsc

N_TASKS = 16
N_MODALITIES = 8
PRIOR_LEN = 16
EMBED_DIM = 768
BATCH = 1024

ROW = PRIOR_LEN * EMBED_DIM
NROWS = 2 * BATCH
NW = 32
ROWS_PER_W = NROWS // NW
CB = 4
NCHUNK = ROWS_PER_W // CB


def _gather_body(table_hbm, idx_hbm, out_hbm, idx_v, buf0, buf1, sem0, sem1):
    nc = 2
    wid = lax.axis_index("s") * nc + lax.axis_index("c")
    base = wid * ROWS_PER_W
    pltpu.sync_copy(idx_hbm.at[wid], idx_v)
    bufs = (buf0, buf1)
    sems = (sem0, sem1)

    pltpu.async_copy(table_hbm.at[idx_v.at[0]], buf0, sem0)
    pltpu.async_copy(table_hbm.at[idx_v.at[1]], buf1, sem1)

    def body(g, carry):
        for s in range(2):
            ch = 2 * g + s
            pltpu.make_async_copy(table_hbm.at[idx_v.at[0]], bufs[s], sems[s]).wait()
            pltpu.sync_copy(bufs[s], out_hbm.at[pl.ds(base + ch * CB, CB)])
            pltpu.async_copy(table_hbm.at[idx_v.at[ch + 2]], bufs[s], sems[s])
        return carry

    lax.fori_loop(0, NCHUNK // 2 - 1, body, 0)

    for s in range(2):
        ch = NCHUNK - 2 + s
        pltpu.make_async_copy(table_hbm.at[idx_v.at[0]], bufs[s], sems[s]).wait()
        pltpu.sync_copy(bufs[s], out_hbm.at[pl.ds(base + ch * CB, CB)])


@jax.jit
def _gather(table, src_idx):
    mesh = plsc.VectorSubcoreMesh(core_axis_name="c", subcore_axis_name="s")
    f = pl.kernel(
        _gather_body,
        out_type=jax.ShapeDtypeStruct((NROWS, ROW), jnp.float32),
        mesh=mesh,
        scratch_types=[
            pltpu.VMEM((NCHUNK, CB), jnp.int32),
            pltpu.VMEM((CB, ROW), jnp.float32),
            pltpu.VMEM((CB, ROW), jnp.float32),
            pltpu.SemaphoreType.DMA,
            pltpu.SemaphoreType.DMA,
        ],
    )
    return f(table, src_idx)


def kernel(task_table, modality_table, task_idx, modality_idx):
    table = jnp.concatenate(
        [task_table.reshape(N_TASKS, ROW),
         modality_table.reshape(N_MODALITIES, ROW)], axis=0)
    src_idx = jnp.stack(
        [task_idx.astype(jnp.int32),
         modality_idx.astype(jnp.int32) + N_TASKS], axis=1)
    src_idx = src_idx.reshape(NW, NCHUNK, CB)
    out = _gather(table, src_idx)
    return out.reshape(BATCH, 2 * PRIOR_LEN, EMBED_DIM)


# speedup vs baseline: 1.7021x; 1.6810x over previous
import jax
import jax.numpy as jnp
from jax import lax
from jax.experimental import pallas as pl
from jax.experimental.pallas import tpu as pltpu
from jax.experimental.pallas import tpu_sc as plsc

N_TASKS = 16
N_MODALITIES = 8
PRIOR_LEN = 16
EMBED_DIM = 768
BATCH = 1024

NW = 32
BPW = BATCH // NW
NB = 8
NCH = BPW // NB


def _gather_body(task_t, mod_t, tidx_h, midx_h, out, tidx_v, midx_v, buf, sem):
    nc = 2
    wid = lax.axis_index("s") * nc + lax.axis_index("c")
    b0 = wid * BPW
    pltpu.sync_copy(tidx_h.at[pl.ds(b0, BPW)], tidx_v)
    pltpu.sync_copy(midx_h.at[pl.ds(b0, BPW)], midx_v)

    def chunk_task(g, carry):
        bb = b0 + g * NB
        pltpu.async_copy(task_t.at[tidx_v.at[pl.ds(g * NB, NB)]], buf, sem).wait()
        pltpu.sync_copy(buf, out.at[pl.ds(bb, NB), pl.ds(0, PRIOR_LEN)])
        return carry

    lax.fori_loop(0, NCH, chunk_task, 0)

    def chunk_mod(g, carry):
        bb = b0 + g * NB
        pltpu.async_copy(mod_t.at[midx_v.at[pl.ds(g * NB, NB)]], buf, sem).wait()
        pltpu.sync_copy(buf, out.at[pl.ds(bb, NB), pl.ds(PRIOR_LEN, PRIOR_LEN)])
        return carry

    lax.fori_loop(0, NCH, chunk_mod, 0)


@jax.jit
def _gather(task_table, modality_table, task_idx, modality_idx):
    mesh = plsc.VectorSubcoreMesh(core_axis_name="c", subcore_axis_name="s")
    f = pl.kernel(
        _gather_body,
        out_type=jax.ShapeDtypeStruct((BATCH, 2 * PRIOR_LEN, EMBED_DIM), jnp.float32),
        mesh=mesh,
        scratch_types=[
            pltpu.VMEM((BPW,), jnp.int32),
            pltpu.VMEM((BPW,), jnp.int32),
            pltpu.VMEM((NB, PRIOR_LEN, EMBED_DIM), jnp.float32),
            pltpu.SemaphoreType.DMA,
        ],
        compiler_params=pltpu.CompilerParams(use_tc_tiling_on_sc=True),
    )
    return f(task_table, modality_table, task_idx, modality_idx)


def kernel(task_table, modality_table, task_idx, modality_idx):
    return _gather(task_table, modality_table,
                   task_idx.astype(jnp.int32), modality_idx.astype(jnp.int32))


# speedup vs baseline: 1.7362x; 1.0200x over previous
import jax
import jax.numpy as jnp
from jax import lax
from jax.experimental import pallas as pl
from jax.experimental.pallas import tpu as pltpu
from jax.experimental.pallas import tpu_sc as plsc

N_TASKS = 16
N_MODALITIES = 8
PRIOR_LEN = 16
EMBED_DIM = 768
BATCH = 1024

NW = 32
BPW = BATCH // NW
NB = 4
NCH = BPW // NB


def _gather_body(task_t, mod_t, tidx_h, midx_h, out, tidx_v, midx_v,
                 buf0, buf1, sem0, sem1):
    nc = 2
    wid = lax.axis_index("s") * nc + lax.axis_index("c")
    b0 = wid * BPW
    pltpu.sync_copy(tidx_h.at[pl.ds(wid * NCH * 8, NCH * 8)], tidx_v)
    pltpu.sync_copy(midx_h.at[pl.ds(wid * NCH * 8, NCH * 8)], midx_v)
    bufs = (buf0, buf1)
    sems = (sem0, sem1)

    def issue(table, idx_v, g, s):
        pltpu.async_copy(table.at[idx_v.at[pl.ds(g * 8, NB)]], bufs[s], sems[s])

    def drain(table, s):
        pltpu.make_async_copy(table.at[tidx_v.at[pl.ds(0, NB)]], bufs[s], sems[s]).wait()

    def store(g, s, tok):
        pltpu.sync_copy(bufs[s], out.at[pl.ds(b0 + g * NB, NB), pl.ds(tok, PRIOR_LEN)])

    issue(task_t, tidx_v, 0, 0)
    issue(task_t, tidx_v, 1, 1)

    def task_body(k, carry):
        for s in range(2):
            g = 2 * k + s
            drain(task_t, s)
            store(g, s, 0)
            issue(task_t, tidx_v, g + 2, s)
        return carry

    lax.fori_loop(0, NCH // 2 - 1, task_body, 0)

    for s in range(2):
        g = NCH - 2 + s
        drain(task_t, s)
        store(g, s, 0)
        issue(mod_t, midx_v, s, s)

    def mod_body(k, carry):
        for s in range(2):
            g = 2 * k + s
            drain(mod_t, s)
            store(g, s, PRIOR_LEN)
            issue(mod_t, midx_v, g + 2, s)
        return carry

    lax.fori_loop(0, NCH // 2 - 1, mod_body, 0)

    for s in range(2):
        g = NCH - 2 + s
        drain(mod_t, s)
        store(g, s, PRIOR_LEN)


@jax.jit
def _gather(task_table, modality_table, task_idx, modality_idx):
    mesh = plsc.VectorSubcoreMesh(core_axis_name="c", subcore_axis_name="s")
    f = pl.kernel(
        _gather_body,
        out_type=jax.ShapeDtypeStruct((BATCH, 2 * PRIOR_LEN, EMBED_DIM), jnp.float32),
        mesh=mesh,
        scratch_types=[
            pltpu.VMEM((NCH * 8,), jnp.int32),
            pltpu.VMEM((NCH * 8,), jnp.int32),
            pltpu.VMEM((NB, PRIOR_LEN, EMBED_DIM), jnp.float32),
            pltpu.VMEM((NB, PRIOR_LEN, EMBED_DIM), jnp.float32),
            pltpu.SemaphoreType.DMA,
            pltpu.SemaphoreType.DMA,
        ],
        compiler_params=pltpu.CompilerParams(use_tc_tiling_on_sc=True),
    )
    return f(task_table, modality_table, task_idx, modality_idx)


def _pad_idx(idx):
    i = idx.astype(jnp.int32).reshape(-1, NB)
    return jnp.pad(i, ((0, 0), (0, 8 - NB))).reshape(-1)


def kernel(task_table, modality_table, task_idx, modality_idx):
    return _gather(task_table, modality_table,
                   _pad_idx(task_idx), _pad_idx(modality_idx))


# speedup vs baseline: 1.8296x; 1.0538x over previous
import jax
import jax.numpy as jnp
from jax import lax
from jax.experimental import pallas as pl
from jax.experimental.pallas import tpu as pltpu
from jax.experimental.pallas import tpu_sc as plsc

N_TASKS = 16
N_MODALITIES = 8
PRIOR_LEN = 16
EMBED_DIM = 768
BATCH = 1024

NW = 32
SLABS = 2 * BATCH
SPW = SLABS // NW
CB = 4
NCH = SPW // CB
BPC = CB // 2


def _gather_body(table, idx_h, out, idx_v, buf0, buf1, sem0, sem1):
    nc = 2
    wid = lax.axis_index("s") * nc + lax.axis_index("c")
    b0 = wid * (BATCH // NW)
    pltpu.sync_copy(idx_h.at[pl.ds(wid * NCH * 8, NCH * 8)], idx_v)
    bufs = (buf0, buf1)
    sems = (sem0, sem1)

    def issue(g, s):
        pltpu.async_copy(table.at[idx_v.at[pl.ds(g * 8, CB)]], bufs[s], sems[s])

    def drain(s):
        pltpu.make_async_copy(table.at[idx_v.at[pl.ds(0, CB)]], bufs[s], sems[s]).wait()

    def store(g, s):
        pltpu.sync_copy(bufs[s].reshape(BPC, 2 * PRIOR_LEN, EMBED_DIM),
                        out.at[pl.ds(b0 + g * BPC, BPC)])

    issue(0, 0)
    issue(1, 1)

    def body(k, carry):
        for s in range(2):
            g = 2 * k + s
            drain(s)
            store(g, s)
            issue(g + 2, s)
        return carry

    lax.fori_loop(0, NCH // 2 - 1, body, 0)

    for s in range(2):
        g = NCH - 2 + s
        drain(s)
        store(g, s)


@jax.jit
def _gather(table, src_idx):
    mesh = plsc.VectorSubcoreMesh(core_axis_name="c", subcore_axis_name="s")
    f = pl.kernel(
        _gather_body,
        out_type=jax.ShapeDtypeStruct((BATCH, 2 * PRIOR_LEN, EMBED_DIM), jnp.float32),
        mesh=mesh,
        scratch_types=[
            pltpu.VMEM((NCH * 8,), jnp.int32),
            pltpu.VMEM((CB, PRIOR_LEN, EMBED_DIM), jnp.float32),
            pltpu.VMEM((CB, PRIOR_LEN, EMBED_DIM), jnp.float32),
            pltpu.SemaphoreType.DMA,
            pltpu.SemaphoreType.DMA,
        ],
        compiler_params=pltpu.CompilerParams(use_tc_tiling_on_sc=True),
    )
    return f(table, src_idx)


def kernel(task_table, modality_table, task_idx, modality_idx):
    table = jnp.concatenate([task_table, modality_table], axis=0)
    src = jnp.stack([task_idx.astype(jnp.int32),
                     modality_idx.astype(jnp.int32) + N_TASKS], axis=1)
    src = src.reshape(-1, CB)
    src = jnp.pad(src, ((0, 0), (0, 8 - CB))).reshape(-1)
    return _gather(table, src)
